# Initial kernel scaffold; baseline (speedup 1.0000x reference)
import jax
import jax.numpy as jnp
from jax.experimental import pallas as pl


def kernel(x, gate_w, w1, w3, w2, sw1, sw3, sw2):
    raise NotImplementedError("write your pallas kernel here")


# speedup vs baseline: 1.0610x; 1.0610x over previous
import jax
import jax.numpy as jnp
from jax.experimental import pallas as pl
from jax.experimental.pallas import tpu as pltpu

DIM = 1024
INTER = 1024
NE = 8
NTOT = NE + 1
TB = 256


def _routing_body(x_ref, gwt_ref, comb_ref):
    logits = jax.lax.dot_general(
        x_ref[...], gwt_ref[...], (((1,), (0,)), ((), ())),
        preferred_element_type=jnp.float32)
    lane = jax.lax.broadcasted_iota(jnp.int32, logits.shape, 1)
    scores = jnp.sqrt(jax.nn.softplus(logits))
    scores = jnp.where(lane < NE, scores, -jnp.inf)
    m1 = jnp.max(scores, axis=1, keepdims=True)
    i1 = jnp.min(jnp.where(scores == m1, lane, 127), axis=1, keepdims=True)
    sel1 = lane == i1
    rest = jnp.where(sel1, -jnp.inf, scores)
    m2 = jnp.max(rest, axis=1, keepdims=True)
    i2 = jnp.min(jnp.where(rest == m2, lane, 127), axis=1, keepdims=True)
    sel2 = lane == i2
    s = m1 + m2
    comb = jnp.where(sel1, m1 / s, 0.0) + jnp.where(sel2, m2 / s, 0.0)
    comb = comb + jnp.where(lane == NE, 1.0, 0.0)
    comb_ref[...] = comb


def _moe_dense_body(comb_ref, xb_ref, w1_ref, w3_ref, w2_ref, out_ref):
    e = pl.program_id(0)
    tb = pl.program_id(1)
    x = xb_ref[...]
    h1 = jnp.dot(x, w1_ref[0], preferred_element_type=jnp.float32)
    h3 = jnp.dot(x, w3_ref[0], preferred_element_type=jnp.float32)
    h = (h1 * (1.0 / (1.0 + jnp.exp(-h1))) * h3).astype(jnp.bfloat16)
    y = jnp.dot(h, w2_ref[0], preferred_element_type=jnp.float32)
    lane16 = jax.lax.broadcasted_iota(jnp.int32, (TB, 16), 1)
    w = jnp.sum(jnp.where(lane16 == e, comb_ref[...], 0.0), axis=1, keepdims=True)
    y = y * w

    @pl.when(e == 0)
    def _():
        out_ref[pl.ds(tb * TB, TB), :] = y

    @pl.when(e > 0)
    def _():
        out_ref[pl.ds(tb * TB, TB), :] += y


def kernel(x, gate_w, w1, w3, w2, sw1, sw3, sw2):
    B, S, D = x.shape
    T = B * S
    xt = x.reshape(T, D)
    nt = T // TB

    gwt = jnp.pad(gate_w, ((0, 16 - NE), (0, 0))).T

    comb = pl.pallas_call(
        _routing_body,
        grid=(nt,),
        in_specs=[
            pl.BlockSpec((TB, DIM), lambda i: (i, 0)),
            pl.BlockSpec((DIM, 16), lambda i: (0, 0)),
        ],
        out_specs=pl.BlockSpec((TB, 16), lambda i: (i, 0)),
        out_shape=jax.ShapeDtypeStruct((T, 16), jnp.float32),
    )(xt, gwt)

    W1 = jnp.concatenate([w1, sw1[None]], 0).transpose(0, 2, 1).astype(jnp.bfloat16)
    W3 = jnp.concatenate([w3, sw3[None]], 0).transpose(0, 2, 1).astype(jnp.bfloat16)
    W2 = jnp.concatenate([w2, sw2[None]], 0).transpose(0, 2, 1).astype(jnp.bfloat16)
    xb = xt.astype(jnp.bfloat16)

    y = pl.pallas_call(
        _moe_dense_body,
        grid=(NTOT, nt),
        in_specs=[
            pl.BlockSpec((TB, 16), lambda e, tb: (tb, 0)),
            pl.BlockSpec((TB, DIM), lambda e, tb: (tb, 0)),
            pl.BlockSpec((1, DIM, INTER), lambda e, tb: (e, 0, 0)),
            pl.BlockSpec((1, DIM, INTER), lambda e, tb: (e, 0, 0)),
            pl.BlockSpec((1, INTER, DIM), lambda e, tb: (e, 0, 0)),
        ],
        out_specs=pl.BlockSpec((T, DIM), lambda e, tb: (0, 0)),
        out_shape=jax.ShapeDtypeStruct((T, DIM), jnp.float32),
        compiler_params=pltpu.CompilerParams(
            dimension_semantics=("arbitrary", "arbitrary")),
    )(comb, xb, W1, W3, W2)

    return y.reshape(B, S, D)
